# Initial kernel scaffold; baseline (speedup 1.0000x reference)
import jax
import jax.numpy as jnp
from jax.experimental import pallas as pl


def kernel(x_idx_batch, context_idx_batch, other_idx_batch, word_emb1, word_emb2):
    raise NotImplementedError("write your pallas kernel here")


# speedup vs baseline: 1.5413x; 1.5413x over previous
import functools

import jax
import jax.numpy as jnp
from jax import lax
from jax.experimental import pallas as pl
from jax.experimental.pallas import tpu as pltpu
from jax.experimental.pallas import tpu_sc as plsc

V = 100000
D = 100
B = 4096
NNEG = 20

NC = 2
NS = 16
L = 16
NW = NC * NS
BPW = B // NW
IPB = 16
NBLK = BPW // IPB
RPB = IPB * NNEG

GPR = 7
GW = 16
RW = GPR * GW
CH = 16
CIDX = CH * GPR
OCH = RPB // CH
XCH = BPW // CH
NG = V * D // GW


def _softplus(z):
    a = jnp.abs(z)
    e = jnp.exp(-a)
    t = e / (e + 2.0)
    t2 = t * t
    p = 2.0 + t2 * (2.0 / 3.0 + t2 * (2.0 / 5.0 + t2 * (2.0 / 7.0 + t2 * (2.0 / 9.0))))
    return jnp.maximum(z, 0.0) + t * p


def _body(xidx_hbm, cidx_hbm, oidx_hbm, xg_hbm, cg_hbm, og_hbm,
          e1g_hbm, e2g_hbm, out_hbm,
          xidx_v, cidx_v, oidx_v, xgi_v, cgi_v, ogi_v,
          xrows, crows, obuf0, obuf1, lossbuf,
          sem_x, sem_c, sem0, sem1):
    wid = lax.axis_index("s") * NC + lax.axis_index("c")

    pltpu.sync_copy(xidx_hbm.at[pl.ds(wid * BPW, BPW)], xidx_v)
    pltpu.sync_copy(cidx_hbm.at[pl.ds(wid * BPW, BPW)], cidx_v)
    pltpu.sync_copy(oidx_hbm.at[wid], oidx_v)
    pltpu.sync_copy(xg_hbm.at[wid], xgi_v)
    pltpu.sync_copy(cg_hbm.at[wid], cgi_v)
    pltpu.sync_copy(og_hbm.at[wid], ogi_v)

    cps_x = [pltpu.async_copy(e1g_hbm.at[xgi_v.at[k]],
                              xrows.at[pl.ds(k * CIDX, CIDX)], sem_x)
             for k in range(XCH)]
    cps_c = [pltpu.async_copy(e2g_hbm.at[cgi_v.at[k]],
                              crows.at[pl.ds(k * CIDX, CIDX)], sem_c)
             for k in range(XCH)]

    bufs = (obuf0, obuf1)
    sems = (sem0, sem1)

    def fire(blk, buf, sem):
        for t in range(OCH):
            pltpu.async_copy(e2g_hbm.at[ogi_v.at[blk * OCH + t]],
                             buf.at[pl.ds(t * CIDX, CIDX)], sem)

    def drain(blk, buf, sem):
        for t in range(OCH):
            pltpu.make_async_copy(e2g_hbm.at[ogi_v.at[blk * OCH + t]],
                                  buf.at[pl.ds(t * CIDX, CIDX)], sem).wait()

    fire(0, obuf0, sem0)
    fire(1, obuf1, sem1)
    for cp in cps_x + cps_c:
        cp.wait()

    iota = lax.broadcasted_iota(jnp.int32, (L,), 0)
    HN = NNEG // 2

    def compute(blk, buf, loss_vec):
        xi = plsc.load_gather(xidx_v, [iota + blk * IPB])
        xbase = (iota + blk * IPB) * RW + ((xi & 3) << 2)

        def half(n0, with_ctx, lv):
            obases = []
            for n in range(n0, n0 + HN):
                oi = plsc.load_gather(oidx_v, [blk * RPB + iota * NNEG + n])
                obases.append(iota * (NNEG * RW) + n * RW + ((oi & 3) << 2))
            if with_ctx:
                ci = plsc.load_gather(cidx_v, [iota + blk * IPB])
                cbase = (iota + blk * IPB) * RW + ((ci & 3) << 2)

            def dbody(d, accs):
                dvec = jnp.full((L,), d, dtype=jnp.int32)
                ax = xbase + dvec
                xcol = plsc.load_gather(xrows, [ax >> 4, ax & 15])
                new = []
                if with_ctx:
                    ac = cbase + dvec
                    ccol = plsc.load_gather(crows, [ac >> 4, ac & 15])
                    new.append(accs[0] + xcol * ccol)
                for k in range(HN):
                    ao = obases[k] + dvec
                    ocol = plsc.load_gather(buf, [ao >> 4, ao & 15])
                    new.append(accs[len(new)] + ocol * xcol)
                return tuple(new)

            nacc = HN + (1 if with_ctx else 0)
            accs0 = tuple(jnp.zeros((L,), jnp.float32) for _ in range(nacc))
            accs = lax.fori_loop(0, D, dbody, accs0)
            if with_ctx:
                lv = lv + _softplus(-accs[0])
                accs = accs[1:]
            for a in accs:
                lv = lv + _softplus(a)
            return lv

        lv = half(0, True, loss_vec)
        return half(HN, False, lv)

    def outer(g2, loss_vec):
        for p in range(2):
            blk = 2 * g2 + p
            drain(blk, bufs[p], sems[p])
            loss_vec = compute(blk, bufs[p], loss_vec)

            @pl.when(blk + 2 < NBLK)
            def _():
                fire(blk + 2, bufs[p], sems[p])
        return loss_vec

    loss_vec = lax.fori_loop(0, NBLK // 2, outer, jnp.zeros((L,), jnp.float32))

    lossbuf[...] = loss_vec * (1.0 / B)
    pltpu.sync_copy(lossbuf, out_hbm.at[wid])


_sc_call = functools.partial(
    pl.kernel,
    out_type=jax.ShapeDtypeStruct((NW, L), jnp.float32),
    mesh=plsc.VectorSubcoreMesh(core_axis_name="c", subcore_axis_name="s"),
    compiler_params=pltpu.CompilerParams(needs_layout_passes=False,
                                         use_tc_tiling_on_sc=False),
    scratch_types=[
        pltpu.VMEM((BPW,), jnp.int32),
        pltpu.VMEM((BPW,), jnp.int32),
        pltpu.VMEM((BPW * NNEG,), jnp.int32),
        pltpu.VMEM((XCH, CIDX), jnp.int32),
        pltpu.VMEM((XCH, CIDX), jnp.int32),
        pltpu.VMEM((NBLK * OCH, CIDX), jnp.int32),
        pltpu.VMEM((BPW * GPR, GW), jnp.float32),
        pltpu.VMEM((BPW * GPR, GW), jnp.float32),
        pltpu.VMEM((RPB * GPR, GW), jnp.float32),
        pltpu.VMEM((RPB * GPR, GW), jnp.float32),
        pltpu.VMEM((L,), jnp.float32),
        pltpu.SemaphoreType.DMA,
        pltpu.SemaphoreType.DMA,
        pltpu.SemaphoreType.DMA,
        pltpu.SemaphoreType.DMA,
    ],
)(_body)


def _granules(idx):
    base = (idx.astype(jnp.int32) * 25) >> 2
    return base[:, None] + jnp.arange(GPR, dtype=jnp.int32)[None, :]


def kernel(x_idx_batch, context_idx_batch, other_idx_batch, word_emb1, word_emb2):
    oflat = other_idx_batch.reshape(-1)
    xg = _granules(x_idx_batch).reshape(NW, XCH, CIDX)
    cg = _granules(context_idx_batch).reshape(NW, XCH, CIDX)
    og = _granules(oflat).reshape(NW, NBLK * OCH, CIDX)
    partial = _sc_call(x_idx_batch, context_idx_batch,
                       oflat.reshape(NW, BPW * NNEG), xg, cg, og,
                       word_emb1.reshape(NG, GW), word_emb2.reshape(NG, GW))
    return jnp.sum(partial)
